# baseline (speedup 1.0000x reference)
import jax
import jax.numpy as jnp
from jax import lax
from jax.experimental import pallas as pl
from jax.experimental.pallas import tpu as pltpu
from jax.experimental.pallas import tpu_sc as plsc

B = 1024
N_VERTS = 5023
N_FACES = 9976
N_LMK = 105
L = 16
NC = 2
NS = 16
NW = NC * NS
BPW = B // NW
NG = 7
LPAD = NG * L
OPAD = 320


def _body(v_hbm, trisf_hbm, fidx_hbm, bary_hbm, out_hbm,
          fidx_v, widx_v, bary_v,
          w0, w1, w2, base0, base1, base2,
          vb0, vb1, ob0, ob1,
          tsem, gsem0, gsem1, osem0, osem1):
    wid = lax.axis_index("s") * NC + lax.axis_index("c")
    b0 = wid * BPW

    iota = lax.iota(jnp.int32, L)

    for g in range(NG):
        fidx_v[pl.ds(g * L, L)] = jnp.zeros((L,), jnp.int32)
    pltpu.sync_copy(fidx_hbm, fidx_v.at[pl.ds(0, N_LMK)])
    for t in range(3 * NG):
        bary_v[pl.ds(t * L, L)] = jnp.zeros((L,), jnp.float32)
    pltpu.sync_copy(bary_hbm, bary_v.at[pl.ds(0, 3 * N_LMK)])

    ws = (w0, w1, w2)
    bases = (base0, base1, base2)
    for k in range(3):
        for g in range(NG):
            widx_v[pl.ds(g * L, L)] = 3 * fidx_v[pl.ds(g * L, L)] + k
        pltpu.async_copy(trisf_hbm.at[widx_v], bases[k], tsem).wait()
        for g in range(NG):
            lidx = g * L + iota
            ws[k][pl.ds(g * L, L)] = plsc.load_gather(bary_v, [3 * lidx + k])

    vbufs = (vb0, vb1)
    obufs = (ob0, ob1)
    gsems = (gsem0, gsem1)
    osems = (osem0, osem1)

    def fire_slab(s, b):
        pltpu.async_copy(v_hbm.at[b], vbufs[s], gsems[s])

    def drain_slab(s, b):
        pltpu.make_async_copy(v_hbm.at[b], vbufs[s], gsems[s]).wait()

    def compute(s):
        vb = vbufs[s]
        for g in range(NG):
            lidx = g * L + iota
            nv = [bases[k][pl.ds(g * L, L)] for k in range(3)]
            wv = [ws[k][pl.ds(g * L, L)] for k in range(3)]
            for c in range(3):
                csplat = jnp.full((L,), c, jnp.int32)
                acc = plsc.load_gather(vb, [nv[0], csplat]) * wv[0]
                acc += plsc.load_gather(vb, [nv[1], csplat]) * wv[1]
                acc += plsc.load_gather(vb, [nv[2], csplat]) * wv[2]
                plsc.store_scatter(obufs[s], [3 * lidx + c], acc)

    def fire_out(s, b):
        pltpu.async_copy(obufs[s].at[pl.ds(0, OPAD)], out_hbm.at[b], osems[s])

    def drain_out(s, b):
        pltpu.make_async_copy(obufs[s].at[pl.ds(0, OPAD)], out_hbm.at[b],
                              osems[s]).wait()

    fire_slab(0, b0)
    fire_slab(1, b0 + 1)

    def loop_body(j, carry):
        for s in range(2):
            i = 2 * j + s
            b = b0 + i

            @pl.when(j > 0)
            def _():
                drain_out(s, b - 2)

            drain_slab(s, b)
            compute(s)
            fire_out(s, b)

            @pl.when(j < BPW // 2 - 1)
            def _():
                fire_slab(s, b + 2)
        return carry

    lax.fori_loop(0, BPW // 2, loop_body, 0)
    drain_out(0, b0 + BPW - 2)
    drain_out(1, b0 + BPW - 1)


@jax.jit
def _lmk_sc(v, tris_flat, lmk_faces_idx, bary_flat):
    mesh = plsc.VectorSubcoreMesh(core_axis_name="c", subcore_axis_name="s",
                                  num_cores=NC, num_subcores=NS)
    return pl.kernel(
        _body,
        out_type=jax.ShapeDtypeStruct((B, OPAD), jnp.float32),
        mesh=mesh,
        compiler_params=pltpu.CompilerParams(needs_layout_passes=False,
                                             use_tc_tiling_on_sc=False),
        scratch_types=[
            pltpu.VMEM((LPAD,), jnp.int32),
            pltpu.VMEM((LPAD,), jnp.int32),
            pltpu.VMEM((3 * LPAD,), jnp.float32),
            pltpu.VMEM((LPAD,), jnp.float32),
            pltpu.VMEM((LPAD,), jnp.float32),
            pltpu.VMEM((LPAD,), jnp.float32),
            pltpu.VMEM((LPAD,), jnp.int32),
            pltpu.VMEM((LPAD,), jnp.int32),
            pltpu.VMEM((LPAD,), jnp.int32),
            pltpu.VMEM((N_VERTS, 3), jnp.float32),
            pltpu.VMEM((N_VERTS, 3), jnp.float32),
            pltpu.VMEM((3 * LPAD,), jnp.float32),
            pltpu.VMEM((3 * LPAD,), jnp.float32),
            pltpu.SemaphoreType.DMA,
            pltpu.SemaphoreType.DMA,
            pltpu.SemaphoreType.DMA,
            pltpu.SemaphoreType.DMA,
            pltpu.SemaphoreType.DMA,
        ],
    )(v, tris_flat, lmk_faces_idx, bary_flat)


def kernel(v, poses, tris, lmk_faces_idx, lmk_bary_coords):
    del poses
    out = _lmk_sc(v,
                  tris.astype(jnp.int32).reshape(3 * N_FACES),
                  lmk_faces_idx.astype(jnp.int32),
                  lmk_bary_coords.astype(jnp.float32).reshape(3 * N_LMK))
    return out[:, :3 * N_LMK].reshape(B, N_LMK, 3)


# speedup vs baseline: 39.3564x; 39.3564x over previous
import jax
import jax.numpy as jnp
from jax import lax
from jax.experimental import pallas as pl
from jax.experimental.pallas import tpu as pltpu
from jax.experimental.pallas import tpu_sc as plsc

B = 1024
N_VERTS = 5023
N_FACES = 9976
N_LMK = 105
L = 16
NC = 2
NS = 16
NW = NC * NS
BPW = B // NW
NG = 7
LPAD = NG * L
OPAD = 320


def _body(v_hbm, trisf_hbm, fidx_hbm, bary_hbm, out_hbm,
          fidx_v, widx_v, bary_v,
          w0, w1, w2, base0, base1, base2,
          vb0, vb1, ob0, ob1,
          tsem, gsem0, gsem1, osem0, osem1):
    wid = lax.axis_index("s") * NC + lax.axis_index("c")
    b0 = wid * BPW

    iota = lax.iota(jnp.int32, L)

    for g in range(NG):
        fidx_v[pl.ds(g * L, L)] = jnp.zeros((L,), jnp.int32)
    pltpu.sync_copy(fidx_hbm, fidx_v.at[pl.ds(0, N_LMK)])
    for t in range(3 * NG):
        bary_v[pl.ds(t * L, L)] = jnp.zeros((L,), jnp.float32)
    pltpu.sync_copy(bary_hbm, bary_v.at[pl.ds(0, 3 * N_LMK)])

    ws = (w0, w1, w2)
    bases = (base0, base1, base2)
    for k in range(3):
        for g in range(NG):
            widx_v[pl.ds(g * L, L)] = 3 * fidx_v[pl.ds(g * L, L)] + k
        pltpu.async_copy(trisf_hbm.at[widx_v], bases[k], tsem).wait()
        for g in range(NG):
            lidx = g * L + iota
            bases[k][pl.ds(g * L, L)] = 3 * bases[k][pl.ds(g * L, L)]
            ws[k][pl.ds(g * L, L)] = plsc.load_gather(bary_v, [3 * lidx + k])

    vbufs = (vb0, vb1)
    obufs = (ob0, ob1)
    gsems = (gsem0, gsem1)
    osems = (osem0, osem1)

    def fire_slab(s, b):
        pltpu.async_copy(v_hbm.at[b], vbufs[s], gsems[s])

    def drain_slab(s, b):
        pltpu.make_async_copy(v_hbm.at[b], vbufs[s], gsems[s]).wait()

    def compute(s):
        vb = vbufs[s]
        for g in range(NG):
            lidx = g * L + iota
            nv = [bases[k][pl.ds(g * L, L)] for k in range(3)]
            wv = [ws[k][pl.ds(g * L, L)] for k in range(3)]
            for c in range(3):
                acc = plsc.load_gather(vb, [nv[0] + c]) * wv[0]
                acc += plsc.load_gather(vb, [nv[1] + c]) * wv[1]
                acc += plsc.load_gather(vb, [nv[2] + c]) * wv[2]
                plsc.store_scatter(obufs[s], [3 * lidx + c], acc)

    def fire_out(s, b):
        pltpu.async_copy(obufs[s].at[pl.ds(0, OPAD)], out_hbm.at[b], osems[s])

    def drain_out(s, b):
        pltpu.make_async_copy(obufs[s].at[pl.ds(0, OPAD)], out_hbm.at[b],
                              osems[s]).wait()

    fire_slab(0, b0)
    fire_slab(1, b0 + 1)

    def loop_body(j, carry):
        for s in range(2):
            i = 2 * j + s
            b = b0 + i

            @pl.when(j > 0)
            def _():
                drain_out(s, b - 2)

            drain_slab(s, b)
            compute(s)
            fire_out(s, b)

            @pl.when(j < BPW // 2 - 1)
            def _():
                fire_slab(s, b + 2)
        return carry

    lax.fori_loop(0, BPW // 2, loop_body, 0)
    drain_out(0, b0 + BPW - 2)
    drain_out(1, b0 + BPW - 1)


@jax.jit
def _lmk_sc(v, tris_flat, lmk_faces_idx, bary_flat):
    mesh = plsc.VectorSubcoreMesh(core_axis_name="c", subcore_axis_name="s",
                                  num_cores=NC, num_subcores=NS)
    return pl.kernel(
        _body,
        out_type=jax.ShapeDtypeStruct((B, OPAD), jnp.float32),
        mesh=mesh,
        compiler_params=pltpu.CompilerParams(needs_layout_passes=False,
                                             use_tc_tiling_on_sc=False),
        scratch_types=[
            pltpu.VMEM((LPAD,), jnp.int32),
            pltpu.VMEM((LPAD,), jnp.int32),
            pltpu.VMEM((3 * LPAD,), jnp.float32),
            pltpu.VMEM((LPAD,), jnp.float32),
            pltpu.VMEM((LPAD,), jnp.float32),
            pltpu.VMEM((LPAD,), jnp.float32),
            pltpu.VMEM((LPAD,), jnp.int32),
            pltpu.VMEM((LPAD,), jnp.int32),
            pltpu.VMEM((LPAD,), jnp.int32),
            pltpu.VMEM((3 * N_VERTS,), jnp.float32),
            pltpu.VMEM((3 * N_VERTS,), jnp.float32),
            pltpu.VMEM((3 * LPAD,), jnp.float32),
            pltpu.VMEM((3 * LPAD,), jnp.float32),
            pltpu.SemaphoreType.DMA,
            pltpu.SemaphoreType.DMA,
            pltpu.SemaphoreType.DMA,
            pltpu.SemaphoreType.DMA,
            pltpu.SemaphoreType.DMA,
        ],
    )(v, tris_flat, lmk_faces_idx, bary_flat)


def kernel(v, poses, tris, lmk_faces_idx, lmk_bary_coords):
    del poses
    out = _lmk_sc(v.reshape(B, 3 * N_VERTS),
                  tris.astype(jnp.int32).reshape(3 * N_FACES),
                  lmk_faces_idx.astype(jnp.int32),
                  lmk_bary_coords.astype(jnp.float32).reshape(3 * N_LMK))
    return out[:, :3 * N_LMK].reshape(B, N_LMK, 3)


# speedup vs baseline: 553.3694x; 14.0605x over previous
import jax
import jax.numpy as jnp
from jax import lax
from jax.experimental import pallas as pl
from jax.experimental.pallas import tpu as pltpu
from jax.experimental.pallas import tpu_sc as plsc

B = 1024
N_VERTS = 5023
N_FACES = 9976
N_LMK = 105
L = 16
NC = 2
NS = 16
NW = NC * NS
NG = 7
LPAD = NG * L
NB = 14
NU = 3 * NB


def _body(vt_hbm, trisf_hbm, fidx_hbm, bary_hbm, out_hbm,
          fidx_v, trisbuf, bary_v,
          w0, w1, w2, base0, base1, base2, tbl,
          gbA, gbB, obA, obB,
          tsem, gsemA, gsemB, osemA, osemB):
    wid = lax.axis_index("s") * NC + lax.axis_index("c")

    iota = lax.iota(jnp.int32, L)

    for g in range(NG):
        fidx_v[pl.ds(g * L, L)] = jnp.zeros((L,), jnp.int32)
    pltpu.sync_copy(fidx_hbm, fidx_v.at[pl.ds(0, N_LMK)])
    for t in range(3 * NG):
        bary_v[pl.ds(t * L, L)] = jnp.zeros((L,), jnp.float32)
    pltpu.sync_copy(bary_hbm, bary_v.at[pl.ds(0, 3 * N_LMK)])
    pltpu.async_copy(trisf_hbm, trisbuf, tsem).wait()

    ws = (w0, w1, w2)
    bases = (base0, base1, base2)
    for k in range(3):
        for g in range(NG):
            lidx = g * L + iota
            fch = fidx_v[pl.ds(g * L, L)]
            bases[k][pl.ds(g * L, L)] = plsc.load_gather(trisbuf, [3 * fch + k])
            ws[k][pl.ds(g * L, L)] = plsc.load_gather(bary_v, [3 * lidx + k])
            pos = (lidx // 8) * 24 + 8 * k + lax.rem(lidx, 8)
            plsc.store_scatter(tbl, [pos], bases[k][pl.ds(g * L, L)])

    def unit(u, gb, ob, gsem, osem):
        c = u // NB
        lb = lax.rem(u, NB)
        return c, lb

    def fire_gather(u, gb, gsem):
        c = u // NB
        lb = lax.rem(u, NB)
        off = pl.multiple_of(24 * lb, 8)
        pltpu.async_copy(vt_hbm.at[c].at[tbl.at[pl.ds(off, 24)]], gb, gsem)

    def drain_gather(u, gb, gsem):
        c = u // NB
        lb = lax.rem(u, NB)
        off = pl.multiple_of(24 * lb, 8)
        pltpu.make_async_copy(vt_hbm.at[c].at[tbl.at[pl.ds(off, 24)]], gb,
                              gsem).wait()

    def compute(u, gb, ob):
        lb = lax.rem(u, NB)
        woff = pl.multiple_of(8 * lb, 8)
        wv = [ws[k][pl.ds(woff, L)] for k in range(3)]
        s0 = [wv[0][dl] for dl in range(8)]
        s1 = [wv[1][dl] for dl in range(8)]
        s2 = [wv[2][dl] for dl in range(8)]

        def chunk(t, carry):
            sl = pl.ds(t * L, L)
            for dl in range(8):
                acc = gb[dl, sl] * s0[dl]
                acc += gb[8 + dl, sl] * s1[dl]
                acc += gb[16 + dl, sl] * s2[dl]
                ob[dl, sl] = acc
            return carry

        lax.fori_loop(0, B // L, chunk, 0)

    def fire_out(u, ob, osem):
        c = u // NB
        lb = lax.rem(u, NB)
        off = pl.multiple_of(8 * lb, 8)
        pltpu.async_copy(ob, out_hbm.at[c].at[pl.ds(off, 8)], osem)

    def drain_out(u, ob, osem):
        c = u // NB
        lb = lax.rem(u, NB)
        off = pl.multiple_of(8 * lb, 8)
        pltpu.make_async_copy(ob, out_hbm.at[c].at[pl.ds(off, 8)],
                              osem).wait()

    uA = wid
    uB = wid + NW
    has_b = wid < NU - NW

    fire_gather(uA, gbA, gsemA)

    @pl.when(has_b)
    def _():
        fire_gather(uB, gbB, gsemB)

    drain_gather(uA, gbA, gsemA)
    compute(uA, gbA, obA)
    fire_out(uA, obA, osemA)

    @pl.when(has_b)
    def _():
        drain_gather(uB, gbB, gsemB)
        compute(uB, gbB, obB)
        fire_out(uB, obB, osemB)

    drain_out(uA, obA, osemA)

    @pl.when(has_b)
    def _():
        drain_out(uB, obB, osemB)


@jax.jit
def _lmk_sc(vt, tris_flat, lmk_faces_idx, bary_flat):
    mesh = plsc.VectorSubcoreMesh(core_axis_name="c", subcore_axis_name="s",
                                  num_cores=NC, num_subcores=NS)
    return pl.kernel(
        _body,
        out_type=jax.ShapeDtypeStruct((3, LPAD, B), jnp.float32),
        mesh=mesh,
        compiler_params=pltpu.CompilerParams(needs_layout_passes=False,
                                             use_tc_tiling_on_sc=True),
        scratch_types=[
            pltpu.VMEM((LPAD,), jnp.int32),
            pltpu.VMEM((3 * N_FACES,), jnp.int32),
            pltpu.VMEM((3 * LPAD,), jnp.float32),
            pltpu.VMEM((2 * L * 8,), jnp.float32),
            pltpu.VMEM((2 * L * 8,), jnp.float32),
            pltpu.VMEM((2 * L * 8,), jnp.float32),
            pltpu.VMEM((LPAD,), jnp.int32),
            pltpu.VMEM((LPAD,), jnp.int32),
            pltpu.VMEM((LPAD,), jnp.int32),
            pltpu.VMEM((NB * 24,), jnp.int32),
            pltpu.VMEM((24, B), jnp.float32),
            pltpu.VMEM((24, B), jnp.float32),
            pltpu.VMEM((8, B), jnp.float32),
            pltpu.VMEM((8, B), jnp.float32),
            pltpu.SemaphoreType.DMA,
            pltpu.SemaphoreType.DMA,
            pltpu.SemaphoreType.DMA,
            pltpu.SemaphoreType.DMA,
            pltpu.SemaphoreType.DMA,
        ],
    )(vt, tris_flat, lmk_faces_idx, bary_flat)


def kernel(v, poses, tris, lmk_faces_idx, lmk_bary_coords):
    del poses
    vt = jnp.transpose(v, (2, 1, 0))
    out_t = _lmk_sc(vt,
                    tris.astype(jnp.int32).reshape(3 * N_FACES),
                    lmk_faces_idx.astype(jnp.int32),
                    lmk_bary_coords.astype(jnp.float32).reshape(3 * N_LMK))
    return jnp.transpose(out_t, (2, 1, 0))[:, :N_LMK, :]
